# baseline (speedup 1.0000x reference)
import functools

import jax
import jax.numpy as jnp
from jax import lax
from jax.experimental import pallas as pl
from jax.experimental.pallas import tpu as pltpu
from jax.experimental.pallas import tpu_sc as plsc

VOCAB = 100000
EMBED_DIM = 64
TOTAL = 4096 * 50

NC = 2
NS = 16
NW = NC * NS
B_PER_W = TOTAL // NW

CHUNK = 400
NCHUNKS = B_PER_W // CHUNK
NBUF = 4
PREF = 2

_mesh = plsc.VectorSubcoreMesh(core_axis_name="c", subcore_axis_name="s")


BATCH = 4096
HIST = 50
B_ROWS = B_PER_W // HIST
C_ROWS = CHUNK // HIST


@functools.partial(
    pl.kernel,
    mesh=_mesh,
    out_type=jax.ShapeDtypeStruct((BATCH, HIST, EMBED_DIM), jnp.float32),
    scratch_types=[
        pltpu.VMEM((NCHUNKS, CHUNK), jnp.int32),
        pltpu.VMEM((NBUF, CHUNK, EMBED_DIM), jnp.float32),
        [pltpu.SemaphoreType.DMA for _ in range(NBUF)],
        [pltpu.SemaphoreType.DMA for _ in range(NBUF)],
    ],
    compiler_params=pltpu.CompilerParams(use_tc_tiling_on_sc=False),
)
def _gather_kernel(idx_hbm, table_hbm, out_hbm, idx_v, rows_v, gsems, wsems):
    wid = lax.axis_index("s") * NC + lax.axis_index("c")
    bbase = wid * B_ROWS

    pltpu.sync_copy(idx_hbm.at[wid], idx_v)

    def write_chunk(j, ci):
        for k in range(C_ROWS):
            pltpu.async_copy(
                rows_v.at[j, pl.ds(k * HIST, HIST)],
                out_hbm.at[bbase + ci * C_ROWS + k],
                wsems[j],
            )

    def drain_write(j):
        pltpu.make_async_copy(
            table_hbm.at[pl.ds(0, CHUNK)], rows_v.at[j], wsems[j]
        ).wait()

    for b in range(PREF):
        pltpu.async_copy(table_hbm.at[idx_v.at[b]], rows_v.at[b], gsems[b])

    @pl.loop(0, NCHUNKS, step=NBUF)
    def _(g):
        for j in range(NBUF):
            ci = g + j
            pltpu.make_async_copy(
                table_hbm.at[idx_v.at[ci]], rows_v.at[j], gsems[j]
            ).wait()
            write_chunk(j, ci)
            nxt = ci + PREF

            @pl.when(nxt < NCHUNKS)
            def _():
                j2 = (j + PREF) % NBUF

                @pl.when(ci >= PREF)
                def _():
                    drain_write(j2)

                pltpu.async_copy(
                    table_hbm.at[idx_v.at[nxt]], rows_v.at[j2], gsems[j2]
                )

    for j in range(NBUF):
        drain_write(j)


def kernel(x, weights):
    idx = x.reshape(NW, NCHUNKS, CHUNK).astype(jnp.int32)
    return _gather_kernel(idx, weights)
